# baseline (speedup 1.0000x reference)
import functools

import numpy as np

import jax
import jax.numpy as jnp
from jax import lax
from jax.experimental import pallas as pl
from jax.experimental.pallas import tpu as pltpu
from jax.experimental.pallas import tpu_sc as plsc

N = 10000
E = 320000
D = 128
EPS = 1e-5

NC = 2
NS = 16
NW = NC * NS
C = 128
NSPLIT = 8

NCHUNK = E // C
CHUNKS_PER_W = NCHUNK // NW
EXTRA = NCHUNK - CHUNKS_PER_W * NW
EPW = CHUNKS_PER_W * C
NP = 10016
INIT_ROWS = NP // 4
OUT_ROWS = 632
OUT_TAIL = N - 15 * OUT_ROWS


def _sc_segment_sum(src_p, dst_p, tables, zeros):
    mesh = plsc.VectorSubcoreMesh(core_axis_name="c", subcore_axis_name="s")
    DS = D // NSPLIT
    NSET = 3
    NTRIP = CHUNKS_PER_W // NSET

    @functools.partial(
        pl.kernel,
        out_type=jax.ShapeDtypeStruct((NC, N, D), jnp.float32),
        mesh=mesh,
        compiler_params=pltpu.CompilerParams(use_tc_tiling_on_sc=False),
        scratch_types=[
            [pltpu.VMEM((C,), jnp.int32)] * NSET,
            [pltpu.VMEM((C,), jnp.int32)] * NSET,
            [[pltpu.VMEM((C, DS), jnp.float32)] * NSPLIT] * NSET,
            [pltpu.VMEM_SHARED((NP, DS), jnp.float32)] * NSPLIT,
            [pltpu.SemaphoreType.DMA] * NSET,
            [pltpu.SemaphoreType.DMA] * NSET,
        ],
    )
    def seg_sum(t0_hbm, t1_hbm, t2_hbm, t3_hbm, t4_hbm, t5_hbm, t6_hbm,
                t7_hbm, src_hbm, dst_hbm, zeros_hbm,
                out_hbm, src_v, dst_v, g, acc, sem_g, sem_s):
        tabs = (t0_hbm, t1_hbm, t2_hbm, t3_hbm, t4_hbm, t5_hbm, t6_hbm,
                t7_hbm)
        cid = lax.axis_index("c")
        sid = lax.axis_index("s")
        wid = sid * NC + cid
        @pl.when(sid < 4)
        def _():
            for i in range(NSPLIT):
                pltpu.sync_copy(zeros_hbm,
                                acc[i].at[pl.ds(sid * INIT_ROWS, INIT_ROWS)])

        plsc.subcore_barrier()

        base = wid * EPW

        def gather_start(off, k):
            pltpu.sync_copy(src_hbm.at[pl.ds(off, C)], src_v[k])
            for i in range(NSPLIT):
                pltpu.async_copy(tabs[i].at[src_v[k]], g[k][i], sem_g[k])

        def gather_wait(k):
            for i in range(NSPLIT):
                pltpu.make_async_copy(tabs[i].at[src_v[k]], g[k][i],
                                      sem_g[k]).wait()

        def scatter_start(off, k):
            pltpu.sync_copy(dst_hbm.at[pl.ds(off, C)], dst_v[k])
            for i in range(NSPLIT):
                pltpu.make_async_copy(
                    g[k][i], acc[i].at[dst_v[k]],
                    sem_s[k]).start(add=True)

        def scatter_wait(k):
            for i in range(NSPLIT):
                pltpu.make_async_copy(
                    g[k][i], acc[i].at[dst_v[k]],
                    sem_s[k]).wait()

        for k in range(NSET):
            gather_start(base + k * C, k)

        @pl.loop(0, NTRIP)
        def _(t):
            j0 = NSET * t
            for k in range(NSET):
                gather_wait(k)
                scatter_start(base + (j0 + k) * C, k)

                @pl.when(t < NTRIP - 1)
                def _():
                    scatter_wait(k)
                    gather_start(base + (j0 + k + NSET) * C, k)

        for k in range(NSET):
            scatter_wait(k)

        @pl.when(wid < EXTRA)
        def _():
            off = (CHUNKS_PER_W * NW + wid) * C
            gather_start(off, 0)
            gather_wait(0)
            scatter_start(off, 0)
            scatter_wait(0)

        plsc.subcore_barrier()

        @pl.when(sid < NS - 1)
        def _():
            for i in range(NSPLIT):
                pltpu.sync_copy(
                    acc[i].at[pl.ds(sid * OUT_ROWS, OUT_ROWS)],
                    out_hbm.at[cid, pl.ds(sid * OUT_ROWS, OUT_ROWS),
                               pl.ds(i * DS, DS)])

        @pl.when(sid == NS - 1)
        def _():
            for i in range(NSPLIT):
                pltpu.sync_copy(
                    acc[i].at[pl.ds((NS - 1) * OUT_ROWS, OUT_TAIL)],
                    out_hbm.at[cid, pl.ds((NS - 1) * OUT_ROWS, OUT_TAIL),
                               pl.ds(i * DS, DS)])

    return seg_sum(*tables, src_p, dst_p, zeros)


R = 1000
NBLK = N // R


def _tc_fused_body(p0_ref, p1_ref, f_ref, w_ref, b_ref, wr_ref, br_ref,
                   g_ref, bt_ref, o_ref, h_all, acc_ref):
    ph = pl.program_id(0)
    i = pl.program_id(1)

    @pl.when(ph == 0)
    def _():
        agg = p0_ref[...] + p1_ref[...]
        h = jnp.dot(agg, w_ref[...], preferred_element_type=jnp.float32)
        h = jnp.maximum(h + b_ref[...], 0.0)
        r = jnp.dot(f_ref[...], wr_ref[...],
                    preferred_element_type=jnp.float32)
        r = jnp.maximum(r + br_ref[...], 0.0)
        h = h + r
        h_all[pl.ds(i * R, R), :] = h

        @pl.when(i == 0)
        def _():
            acc_ref[...] = jnp.zeros_like(acc_ref)

        acc_ref[0:1, :] += jnp.sum(h, axis=0, keepdims=True)
        acc_ref[1:2, :] += jnp.sum(h * h, axis=0, keepdims=True)

    @pl.when(ph == 1)
    def _():
        mean = acc_ref[0:1, :] * (1.0 / N)
        var = acc_ref[1:2, :] * (1.0 / N) - mean * mean
        inv = lax.rsqrt(var + EPS)
        h = h_all[pl.ds(i * R, R), :]
        o_ref[...] = (h - mean) * (inv * g_ref[...]) + bt_ref[...]


def kernel(feats, edge_index, W, b, W_res, b_res, gamma, beta):
    src_p = edge_index[0].astype(jnp.int32)
    dst_p = edge_index[1].astype(jnp.int32)
    zeros = jnp.zeros((INIT_ROWS, D // NSPLIT), jnp.float32)

    tables = [feats[:, k * (D // NSPLIT):(k + 1) * (D // NSPLIT)]
              for k in range(NSPLIT)]
    parts = _sc_segment_sum(src_p, dst_p, tables, zeros)
    p0, p1 = parts[0], parts[1]

    blk = lambda ph, i: (i * (1 - ph), 0)
    out_blk = lambda ph, i: (i, 0)
    full = lambda ph, i: (0, 0)
    out = pl.pallas_call(
        _tc_fused_body,
        grid=(2, NBLK),
        in_specs=[
            pl.BlockSpec((R, D), blk),
            pl.BlockSpec((R, D), blk),
            pl.BlockSpec((R, D), blk),
            pl.BlockSpec((D, D), full),
            pl.BlockSpec((1, D), full),
            pl.BlockSpec((D, D), full),
            pl.BlockSpec((1, D), full),
            pl.BlockSpec((1, D), full),
            pl.BlockSpec((1, D), full),
        ],
        out_specs=pl.BlockSpec((R, D), out_blk),
        out_shape=jax.ShapeDtypeStruct((N, D), jnp.float32),
        scratch_shapes=[
            pltpu.VMEM((N, D), jnp.float32),
            pltpu.VMEM((2, D), jnp.float32),
        ],
    )(p0, p1, feats, W, b.reshape(1, D), W_res, b_res.reshape(1, D),
      gamma.reshape(1, D), beta.reshape(1, D))
    return out


# speedup vs baseline: 1.2553x; 1.2553x over previous
import functools

import numpy as np

import jax
import jax.numpy as jnp
from jax import lax
from jax.experimental import pallas as pl
from jax.experimental.pallas import tpu as pltpu
from jax.experimental.pallas import tpu_sc as plsc

N = 10000
E = 320000
D = 128
EPS = 1e-5

NC = 2
NS = 16
NW = NC * NS
C = 128
NSPLIT = 4

NCHUNK = E // C
CHUNKS_PER_W = NCHUNK // NW
EXTRA = NCHUNK - CHUNKS_PER_W * NW
EPW = CHUNKS_PER_W * C
NP = 10016
INIT_ROWS = NP // 4
OUT_ROWS = 632
OUT_TAIL = N - 15 * OUT_ROWS


def _sc_segment_sum(src_p, dst_p, tables, zeros):
    mesh = plsc.VectorSubcoreMesh(core_axis_name="c", subcore_axis_name="s")
    DS = D // NSPLIT
    NSET = 3
    NTRIP = CHUNKS_PER_W // NSET

    @functools.partial(
        pl.kernel,
        out_type=jax.ShapeDtypeStruct((NC, N, D), jnp.float32),
        mesh=mesh,
        compiler_params=pltpu.CompilerParams(use_tc_tiling_on_sc=False),
        scratch_types=[
            [pltpu.VMEM((C,), jnp.int32)] * NSET,
            [pltpu.VMEM((C,), jnp.int32)] * NSET,
            [[pltpu.VMEM((C, DS), jnp.float32)] * NSPLIT] * NSET,
            [pltpu.VMEM_SHARED((NP, DS), jnp.float32)] * NSPLIT,
            [pltpu.SemaphoreType.DMA] * NSET,
            [pltpu.SemaphoreType.DMA] * NSET,
        ],
    )
    def seg_sum(t0_hbm, t1_hbm, t2_hbm, t3_hbm, src_hbm, dst_hbm, zeros_hbm,
                out_hbm, src_v, dst_v, g, acc, sem_g, sem_s):
        tabs = (t0_hbm, t1_hbm, t2_hbm, t3_hbm)
        cid = lax.axis_index("c")
        sid = lax.axis_index("s")
        wid = sid * NC + cid
        @pl.when(sid < 4)
        def _():
            for i in range(NSPLIT):
                pltpu.sync_copy(zeros_hbm,
                                acc[i].at[pl.ds(sid * INIT_ROWS, INIT_ROWS)])

        plsc.subcore_barrier()

        base = wid * EPW

        def gather_start(off, k):
            pltpu.sync_copy(src_hbm.at[pl.ds(off, C)], src_v[k])
            for i in range(NSPLIT):
                pltpu.async_copy(tabs[i].at[src_v[k]], g[k][i], sem_g[k])

        def gather_wait(k):
            for i in range(NSPLIT):
                pltpu.make_async_copy(tabs[i].at[src_v[k]], g[k][i],
                                      sem_g[k]).wait()

        def scatter_start(off, k):
            pltpu.sync_copy(dst_hbm.at[pl.ds(off, C)], dst_v[k])
            for i in range(NSPLIT):
                pltpu.make_async_copy(
                    g[k][i], acc[i].at[dst_v[k]],
                    sem_s[k]).start(add=True)

        def scatter_wait(k):
            for i in range(NSPLIT):
                pltpu.make_async_copy(
                    g[k][i], acc[i].at[dst_v[k]],
                    sem_s[k]).wait()

        for k in range(NSET):
            gather_start(base + k * C, k)

        @pl.loop(0, NTRIP)
        def _(t):
            j0 = NSET * t
            for k in range(NSET):
                gather_wait(k)
                scatter_start(base + (j0 + k) * C, k)

                @pl.when(t < NTRIP - 1)
                def _():
                    scatter_wait(k)
                    gather_start(base + (j0 + k + NSET) * C, k)

        for k in range(NSET):
            scatter_wait(k)

        @pl.when(wid < EXTRA)
        def _():
            off = (CHUNKS_PER_W * NW + wid) * C
            gather_start(off, 0)
            gather_wait(0)
            scatter_start(off, 0)
            scatter_wait(0)

        plsc.subcore_barrier()

        @pl.when(sid < NS - 1)
        def _():
            for i in range(NSPLIT):
                pltpu.sync_copy(
                    acc[i].at[pl.ds(sid * OUT_ROWS, OUT_ROWS)],
                    out_hbm.at[cid, pl.ds(sid * OUT_ROWS, OUT_ROWS),
                               pl.ds(i * DS, DS)])

        @pl.when(sid == NS - 1)
        def _():
            for i in range(NSPLIT):
                pltpu.sync_copy(
                    acc[i].at[pl.ds((NS - 1) * OUT_ROWS, OUT_TAIL)],
                    out_hbm.at[cid, pl.ds((NS - 1) * OUT_ROWS, OUT_TAIL),
                               pl.ds(i * DS, DS)])

    return seg_sum(*tables, src_p, dst_p, zeros)


R = 1000
NBLK = N // R


def _tc_fused_body(p0_ref, p1_ref, f_ref, w_ref, b_ref, wr_ref, br_ref,
                   g_ref, bt_ref, o_ref, h_all, acc_ref):
    ph = pl.program_id(0)
    i = pl.program_id(1)

    @pl.when(ph == 0)
    def _():
        agg = p0_ref[...] + p1_ref[...]
        h = jnp.dot(agg, w_ref[...], preferred_element_type=jnp.float32)
        h = jnp.maximum(h + b_ref[...], 0.0)
        r = jnp.dot(f_ref[...], wr_ref[...],
                    preferred_element_type=jnp.float32)
        r = jnp.maximum(r + br_ref[...], 0.0)
        h = h + r
        h_all[pl.ds(i * R, R), :] = h

        @pl.when(i == 0)
        def _():
            acc_ref[...] = jnp.zeros_like(acc_ref)

        acc_ref[0:1, :] += jnp.sum(h, axis=0, keepdims=True)
        acc_ref[1:2, :] += jnp.sum(h * h, axis=0, keepdims=True)

    @pl.when(ph == 1)
    def _():
        mean = acc_ref[0:1, :] * (1.0 / N)
        var = acc_ref[1:2, :] * (1.0 / N) - mean * mean
        inv = lax.rsqrt(var + EPS)
        h = h_all[pl.ds(i * R, R), :]
        o_ref[...] = (h - mean) * (inv * g_ref[...]) + bt_ref[...]


def kernel(feats, edge_index, W, b, W_res, b_res, gamma, beta):
    src_p = edge_index[0].astype(jnp.int32)
    dst_p = edge_index[1].astype(jnp.int32)
    zeros = jnp.zeros((INIT_ROWS, D // NSPLIT), jnp.float32)

    tables = [feats[:, k * (D // NSPLIT):(k + 1) * (D // NSPLIT)]
              for k in range(NSPLIT)]
    parts = _sc_segment_sum(src_p, dst_p, tables, zeros)
    p0, p1 = parts[0], parts[1]

    blk = lambda ph, i: (i * (1 - ph), 0)
    out_blk = lambda ph, i: (i, 0)
    full = lambda ph, i: (0, 0)
    out = pl.pallas_call(
        _tc_fused_body,
        grid=(2, NBLK),
        in_specs=[
            pl.BlockSpec((R, D), blk),
            pl.BlockSpec((R, D), blk),
            pl.BlockSpec((R, D), blk),
            pl.BlockSpec((D, D), full),
            pl.BlockSpec((1, D), full),
            pl.BlockSpec((D, D), full),
            pl.BlockSpec((1, D), full),
            pl.BlockSpec((1, D), full),
            pl.BlockSpec((1, D), full),
        ],
        out_specs=pl.BlockSpec((R, D), out_blk),
        out_shape=jax.ShapeDtypeStruct((N, D), jnp.float32),
        scratch_shapes=[
            pltpu.VMEM((N, D), jnp.float32),
            pltpu.VMEM((2, D), jnp.float32),
        ],
    )(p0, p1, feats, W, b.reshape(1, D), W_res, b_res.reshape(1, D),
      gamma.reshape(1, D), beta.reshape(1, D))
    return out
